# baseline (speedup 1.0000x reference)
import functools

import jax
import jax.numpy as jnp
from jax import lax
from jax.experimental import pallas as pl
from jax.experimental.pallas import tpu as pltpu
from jax.experimental.pallas import tpu_sc as plsc

_T = 256
_NW = 32
_NC = 8
_MW = 16


def _router_body(x_ref, w_ref, logits_ref, idx_ref, xs_ref, pos_ref, meta_ref):
    s, e = logits_ref.shape
    cs = s // _NC
    nt = s // _T
    f32 = jnp.float32
    hi_p = jax.lax.Precision.HIGHEST

    logits = jnp.dot(x_ref[...], w_ref[...], preferred_element_type=f32)
    logits_ref[...] = logits
    m = jnp.max(logits, axis=-1, keepdims=True)
    ssum = jnp.sum(jnp.exp(logits - m), axis=-1, keepdims=True)
    pmax = 1.0 / ssum
    lane = lax.broadcasted_iota(jnp.int32, (s, e), 1)
    eidx = jnp.min(jnp.where(logits == m, lane, e), axis=-1, keepdims=True)
    idx_ref[...] = eidx
    xs_ref[...] = x_ref[...] * pmax

    oh = (lane == eidx).astype(f32)
    ohc = oh.reshape(_NC, cs, e)
    chunk_counts = jnp.sum(ohc, axis=1)
    counts = jnp.sum(chunk_counts, axis=0, keepdims=True)
    i8 = lax.broadcasted_iota(jnp.int32, (_NC, _NC), 0)
    j8 = lax.broadcasted_iota(jnp.int32, (_NC, _NC), 1)
    u8 = (i8 < j8).astype(f32)
    l8 = (i8 > j8).astype(f32)
    eye8 = (i8 == j8).astype(f32)
    starts = jnp.dot(counts, u8, preferred_element_type=f32, precision=hi_p)
    ends = starts + counts
    posbase = jnp.dot(l8, chunk_counts, preferred_element_type=f32,
                      precision=hi_p) + starts
    ic = lax.broadcasted_iota(jnp.int32, (cs, cs), 0)
    jc = lax.broadcasted_iota(jnp.int32, (cs, cs), 1)
    lc = (ic > jc).astype(f32)
    pieces = []
    for c in range(_NC):
        within = jnp.dot(lc, ohc[c], preferred_element_type=f32)
        pieces.append(within + posbase[c:c + 1, :])
    posfull = jnp.concatenate(pieces, axis=0)
    pos_ref[...] = jnp.sum(posfull * oh, axis=1,
                           keepdims=True).astype(jnp.int32)

    starts_col = jnp.sum(jnp.broadcast_to(starts, (e, e)) * eye8,
                         axis=1, keepdims=True)
    ends_col = jnp.sum(jnp.broadcast_to(ends, (e, e)) * eye8,
                       axis=1, keepdims=True)
    t_lo = lax.broadcasted_iota(jnp.int32, (nt, e), 0).astype(f32) * _T
    endsb = jnp.broadcast_to(ends, (nt, e))
    ef_col = jnp.sum((endsb <= t_lo).astype(f32), axis=1, keepdims=True)
    el_col = jnp.sum((endsb <= t_lo + (_T - 1)).astype(f32),
                     axis=1, keepdims=True)
    cnt_col = el_col - ef_col + 1.0
    base_col = jnp.dot(l8[:nt, :nt], cnt_col,
                       preferred_element_type=f32, precision=hi_p)
    total = jnp.sum(cnt_col, axis=0, keepdims=True)
    wrow = lax.broadcasted_iota(jnp.int32, (1, _MW), 1).astype(f32)
    wlane = jnp.broadcast_to(wrow, (nt, _MW))
    baseb = jnp.broadcast_to(base_col, (nt, _MW))
    tw = jnp.clip(
        jnp.sum((baseb <= wlane).astype(f32), axis=0, keepdims=True) - 1.0,
        0.0, nt - 1.0)
    tmatch = (lax.broadcasted_iota(jnp.int32, (nt, _MW), 0).astype(f32)
              == jnp.broadcast_to(tw, (nt, _MW))).astype(f32)
    base_at = jnp.sum(baseb * tmatch, axis=0, keepdims=True)
    ef_at = jnp.sum(jnp.broadcast_to(ef_col, (nt, _MW)) * tmatch,
                    axis=0, keepdims=True)
    k = wrow - base_at
    ew = jnp.clip(ef_at + k, 0.0, e - 1.0)
    ematch = (lax.broadcasted_iota(jnp.int32, (e, _MW), 0).astype(f32)
              == jnp.broadcast_to(ew, (e, _MW))).astype(f32)
    starts_at = jnp.sum(jnp.broadcast_to(starts_col, (e, _MW)) * ematch,
                        axis=0, keepdims=True)
    ends_at = jnp.sum(jnp.broadcast_to(ends_col, (e, _MW)) * ematch,
                      axis=0, keepdims=True)
    valid = wrow < total
    lo = jnp.clip(starts_at - tw * _T, 0.0, float(_T))
    hi = jnp.clip(ends_at - tw * _T, 0.0, float(_T))
    lo = jnp.where(valid, lo, 0.0)
    hi = jnp.where(valid, hi, 0.0)

    i16 = lax.broadcasted_iota(jnp.int32, (_MW, _MW), 0)
    j16 = lax.broadcasted_iota(jnp.int32, (_MW, _MW), 1)
    eye16 = (i16 == j16).astype(f32)
    sh16 = (i16 == (j16 - 1)).astype(f32)
    le16 = (i16 <= j16).astype(f32)
    ew_lv = jnp.sum(ew * (wrow == total - 1.0).astype(f32),
                    axis=1, keepdims=True)
    ew2 = jnp.where(valid, ew, ew_lv)
    ew_prev = jnp.dot(ew2, sh16, preferred_element_type=f32, precision=hi_p)
    change = ((ew2 != ew_prev) & (wrow > 0.0)).astype(f32)
    run_id = jnp.dot(change, le16, preferred_element_type=f32, precision=hi_p)
    first_run = jnp.maximum(change, (wrow == 0.0).astype(f32))
    rm1 = jnp.sum(change, axis=1, keepdims=True)
    fetch_flag = first_run * (run_id < rm1).astype(f32)
    match = ((i16.astype(f32) == jnp.broadcast_to(run_id, (_MW, _MW)))
             * jnp.broadcast_to(first_run, (_MW, _MW)))
    re_col = jnp.sum(match * jnp.broadcast_to(ew2, (_MW, _MW)),
                     axis=1, keepdims=True)
    re_row = jnp.sum(jnp.broadcast_to(re_col, (_MW, _MW)) * eye16,
                     axis=0, keepdims=True)
    meta_ref[...] = jnp.concatenate(
        [tw, ew2, lo, hi, first_run, run_id, re_row, fetch_flag],
        axis=0).astype(jnp.int32)


def _route(x, w_router, interpret=False):
    s, d = x.shape
    e = w_router.shape[-1]
    return pl.pallas_call(
        _router_body,
        out_shape=[
            jax.ShapeDtypeStruct((s, e), jnp.float32),
            jax.ShapeDtypeStruct((s, 1), jnp.int32),
            jax.ShapeDtypeStruct((s, d), jnp.float32),
            jax.ShapeDtypeStruct((s, 1), jnp.int32),
            jax.ShapeDtypeStruct((8, _MW), jnp.int32),
        ],
        interpret=interpret,
    )(x, w_router)


_NF = 1


def _ffn_body(meta_ref, xs_ref, wi_hbm, wo_hbm, y_ref,
              wib, wob, semi, semo):
    w = pl.program_id(0)
    run = meta_ref[5, w]
    par = lax.rem(run, 2)

    def _fetch_copies(expert, buf):
        d_h = wi_hbm.shape[1] // 2
        f_h = wo_hbm.shape[1] // 2
        return [
            pltpu.make_async_copy(wi_hbm.at[expert, pl.ds(0, d_h)],
                                  wib.at[buf, pl.ds(0, d_h)], semi.at[buf]),
            pltpu.make_async_copy(wi_hbm.at[expert, pl.ds(d_h, d_h)],
                                  wib.at[buf, pl.ds(d_h, d_h)], semi.at[buf]),
            pltpu.make_async_copy(wo_hbm.at[expert, pl.ds(0, f_h)],
                                  wob.at[buf, pl.ds(0, f_h)], semo.at[buf]),
            pltpu.make_async_copy(wo_hbm.at[expert, pl.ds(f_h, f_h)],
                                  wob.at[buf, pl.ds(f_h, f_h)], semo.at[buf]),
        ]

    @pl.when(w == 0)
    def _start():
        y_ref[...] = jnp.zeros_like(y_ref)
        cps = _fetch_copies(meta_ref[6, 0], 0)
        for cp in cps:
            cp.start()
        for cp in cps:
            cp.wait()

    @pl.when(meta_ref[7, w] == 1)
    def _prefetch():
        for cp in _fetch_copies(meta_ref[6, run + 1], 1 - par):
            cp.start()

    @pl.when((meta_ref[4, w] == 1) & (run > 0))
    def _wait():
        for cp in _fetch_copies(meta_ref[6, run], par):
            cp.wait()

    t = meta_ref[0, w]
    x = xs_ref[pl.ds(t * _T, _T), :]
    h = jnp.maximum(
        jnp.dot(x, wib[par], preferred_element_type=jnp.float32), 0.0)
    y = jnp.dot(h, wob[par], preferred_element_type=jnp.float32)
    rows = lax.broadcasted_iota(jnp.int32, (y.shape[0], 1), 0)
    mask = (rows >= meta_ref[2, w]) & (rows < meta_ref[3, w])
    y_ref[pl.ds(t * _T, _T), :] += jnp.where(mask, y, 0.0)


def _grouped_ffn(xs, wi, wo, meta, interpret=False):
    s, d = xs.shape
    e, _, f = wi.shape
    n_items = (s // _T) + e - 1
    grid_spec = pltpu.PrefetchScalarGridSpec(
        num_scalar_prefetch=1,
        grid=(n_items,),
        in_specs=[
            pl.BlockSpec((s, d), lambda w, meta: (0, 0)),
            pl.BlockSpec(memory_space=pl.ANY),
            pl.BlockSpec(memory_space=pl.ANY),
        ],
        out_specs=pl.BlockSpec((s, d), lambda w, meta: (0, 0)),
        scratch_shapes=[
            pltpu.VMEM((2, d, f), jnp.float32),
            pltpu.VMEM((2, f, d), jnp.float32),
            pltpu.SemaphoreType.DMA((2,)),
            pltpu.SemaphoreType.DMA((2,)),
        ],
    )
    return pl.pallas_call(
        _ffn_body,
        grid_spec=grid_spec,
        out_shape=jax.ShapeDtypeStruct((s, d), jnp.float32),
        compiler_params=pltpu.CompilerParams(
            dimension_semantics=("arbitrary",)),
        interpret=interpret,
    )(meta, xs, wi, wo)


def _sc_scatter_rows(src, pos):
    s, d = src.shape
    rpw = s // _NW
    mesh = plsc.VectorSubcoreMesh(core_axis_name="c", subcore_axis_name="s")

    @functools.partial(
        pl.kernel, mesh=mesh,
        out_type=jax.ShapeDtypeStruct((s, d), jnp.float32),
        scratch_types=[
            pltpu.VMEM((rpw,), jnp.int32),
            pltpu.VMEM((rpw, d), jnp.float32),
            pltpu.SemaphoreType.DMA,
        ])
    def k(src_hbm, idx_hbm, out_hbm, idx_v, rows_v, sem):
        wid = lax.axis_index("s") * 2 + lax.axis_index("c")
        base = wid * rpw
        pltpu.sync_copy(idx_hbm.at[pl.ds(base, rpw)], idx_v)
        pltpu.sync_copy(src_hbm.at[pl.ds(base, rpw)], rows_v)
        pltpu.async_copy(rows_v, out_hbm.at[idx_v], sem).wait()

    return k(src, pos)


def _sc_gather_rows(src, pos):
    s, d = src.shape
    rpw = s // _NW
    mesh = plsc.VectorSubcoreMesh(core_axis_name="c", subcore_axis_name="s")

    @functools.partial(
        pl.kernel, mesh=mesh,
        out_type=jax.ShapeDtypeStruct((s, d), jnp.float32),
        scratch_types=[
            pltpu.VMEM((rpw,), jnp.int32),
            pltpu.VMEM((rpw, d), jnp.float32),
            pltpu.SemaphoreType.DMA,
        ])
    def k(src_hbm, idx_hbm, out_hbm, idx_v, rows_v, sem):
        wid = lax.axis_index("s") * 2 + lax.axis_index("c")
        base = wid * rpw
        pltpu.sync_copy(idx_hbm.at[pl.ds(base, rpw)], idx_v)
        pltpu.async_copy(src_hbm.at[idx_v], rows_v, sem).wait()
        pltpu.sync_copy(rows_v, out_hbm.at[pl.ds(base, rpw)])

    return k(src, pos)


def kernel(hidden_states, W_router, wi, wo):
    b, s, d = hidden_states.shape
    e = W_router.shape[-1]
    x = hidden_states.reshape(b * s, d)

    logits, idx2, x_scaled, pos2, meta = _route(x, W_router)
    pos = pos2.reshape(b * s)

    xs = _sc_scatter_rows(x_scaled, pos)
    ys = _grouped_ffn(xs, wi, wo, meta)
    out = _sc_gather_rows(ys, pos)

    return (out.reshape(b, s, d),
            logits.reshape(b, s, e),
            idx2.reshape(b, s))


# speedup vs baseline: 1.0245x; 1.0245x over previous
import functools

import jax
import jax.numpy as jnp
from jax import lax
from jax.experimental import pallas as pl
from jax.experimental.pallas import tpu as pltpu
from jax.experimental.pallas import tpu_sc as plsc

_T = 256
_NW = 32
_NC = 8
_MW = 16


def _router_body(x_ref, w_ref, logits_ref, idx_ref, xs_ref, pos_ref, meta_ref):
    s, e = logits_ref.shape
    cs = s // _NC
    nt = s // _T
    f32 = jnp.float32
    hi_p = jax.lax.Precision.HIGHEST

    logits = jnp.dot(x_ref[...], w_ref[...], preferred_element_type=f32)
    logits_ref[...] = logits
    m = jnp.max(logits, axis=-1, keepdims=True)
    ssum = jnp.sum(jnp.exp(logits - m), axis=-1, keepdims=True)
    pmax = 1.0 / ssum
    lane = lax.broadcasted_iota(jnp.int32, (s, e), 1)
    eidx = jnp.min(jnp.where(logits == m, lane, e), axis=-1, keepdims=True)
    idx_ref[...] = eidx
    xs_ref[...] = x_ref[...] * pmax

    oh = (lane == eidx).astype(f32)
    ohc = oh.reshape(_NC, cs, e)
    chunk_counts = jnp.sum(ohc, axis=1)
    counts = jnp.sum(chunk_counts, axis=0, keepdims=True)
    i8 = lax.broadcasted_iota(jnp.int32, (_NC, _NC), 0)
    j8 = lax.broadcasted_iota(jnp.int32, (_NC, _NC), 1)
    u8 = (i8 < j8).astype(f32)
    l8 = (i8 > j8).astype(f32)
    eye8 = (i8 == j8).astype(f32)
    starts = jnp.dot(counts, u8, preferred_element_type=f32, precision=hi_p)
    ends = starts + counts
    posbase = jnp.dot(l8, chunk_counts, preferred_element_type=f32,
                      precision=hi_p) + starts
    ic = lax.broadcasted_iota(jnp.int32, (cs, cs), 0)
    jc = lax.broadcasted_iota(jnp.int32, (cs, cs), 1)
    lc = (ic > jc).astype(f32)
    pieces = []
    for c in range(_NC):
        within = jnp.dot(lc, ohc[c], preferred_element_type=f32)
        pieces.append(within + posbase[c:c + 1, :])
    posfull = jnp.concatenate(pieces, axis=0)
    pos_col = jnp.sum(posfull * oh, axis=1, keepdims=True)
    i7 = lax.broadcasted_iota(jnp.int32, (128, 128), 0)
    j7 = lax.broadcasted_iota(jnp.int32, (128, 128), 1)
    eye128 = (i7 == j7).astype(f32)
    prow = []
    for g in range(s // 128):
        seg = jnp.broadcast_to(pos_col[g * 128:(g + 1) * 128, :], (128, 128))
        prow.append(jnp.sum(seg * eye128, axis=0, keepdims=True))
    pos_ref[...] = jnp.concatenate(prow, axis=0).astype(jnp.int32)

    starts_col = jnp.sum(jnp.broadcast_to(starts, (e, e)) * eye8,
                         axis=1, keepdims=True)
    ends_col = jnp.sum(jnp.broadcast_to(ends, (e, e)) * eye8,
                       axis=1, keepdims=True)
    t_lo = lax.broadcasted_iota(jnp.int32, (nt, e), 0).astype(f32) * _T
    endsb = jnp.broadcast_to(ends, (nt, e))
    ef_col = jnp.sum((endsb <= t_lo).astype(f32), axis=1, keepdims=True)
    el_col = jnp.sum((endsb <= t_lo + (_T - 1)).astype(f32),
                     axis=1, keepdims=True)
    cnt_col = el_col - ef_col + 1.0
    base_col = jnp.dot(l8[:nt, :nt], cnt_col,
                       preferred_element_type=f32, precision=hi_p)
    total = jnp.sum(cnt_col, axis=0, keepdims=True)
    wrow = lax.broadcasted_iota(jnp.int32, (1, _MW), 1).astype(f32)
    wlane = jnp.broadcast_to(wrow, (nt, _MW))
    baseb = jnp.broadcast_to(base_col, (nt, _MW))
    tw = jnp.clip(
        jnp.sum((baseb <= wlane).astype(f32), axis=0, keepdims=True) - 1.0,
        0.0, nt - 1.0)
    tmatch = (lax.broadcasted_iota(jnp.int32, (nt, _MW), 0).astype(f32)
              == jnp.broadcast_to(tw, (nt, _MW))).astype(f32)
    base_at = jnp.sum(baseb * tmatch, axis=0, keepdims=True)
    ef_at = jnp.sum(jnp.broadcast_to(ef_col, (nt, _MW)) * tmatch,
                    axis=0, keepdims=True)
    k = wrow - base_at
    ew = jnp.clip(ef_at + k, 0.0, e - 1.0)
    ematch = (lax.broadcasted_iota(jnp.int32, (e, _MW), 0).astype(f32)
              == jnp.broadcast_to(ew, (e, _MW))).astype(f32)
    starts_at = jnp.sum(jnp.broadcast_to(starts_col, (e, _MW)) * ematch,
                        axis=0, keepdims=True)
    ends_at = jnp.sum(jnp.broadcast_to(ends_col, (e, _MW)) * ematch,
                      axis=0, keepdims=True)
    valid = wrow < total
    lo = jnp.clip(starts_at - tw * _T, 0.0, float(_T))
    hi = jnp.clip(ends_at - tw * _T, 0.0, float(_T))
    lo = jnp.where(valid, lo, 0.0)
    hi = jnp.where(valid, hi, 0.0)

    i16 = lax.broadcasted_iota(jnp.int32, (_MW, _MW), 0)
    j16 = lax.broadcasted_iota(jnp.int32, (_MW, _MW), 1)
    eye16 = (i16 == j16).astype(f32)
    sh16 = (i16 == (j16 - 1)).astype(f32)
    le16 = (i16 <= j16).astype(f32)
    ew_lv = jnp.sum(ew * (wrow == total - 1.0).astype(f32),
                    axis=1, keepdims=True)
    ew2 = jnp.where(valid, ew, ew_lv)
    ew_prev = jnp.dot(ew2, sh16, preferred_element_type=f32, precision=hi_p)
    change = ((ew2 != ew_prev) & (wrow > 0.0)).astype(f32)
    run_id = jnp.dot(change, le16, preferred_element_type=f32, precision=hi_p)
    first_run = jnp.maximum(change, (wrow == 0.0).astype(f32))
    rm1 = jnp.sum(change, axis=1, keepdims=True)
    fetch_flag = first_run * (run_id < rm1).astype(f32)
    match = ((i16.astype(f32) == jnp.broadcast_to(run_id, (_MW, _MW)))
             * jnp.broadcast_to(first_run, (_MW, _MW)))
    re_col = jnp.sum(match * jnp.broadcast_to(ew2, (_MW, _MW)),
                     axis=1, keepdims=True)
    re_row = jnp.sum(jnp.broadcast_to(re_col, (_MW, _MW)) * eye16,
                     axis=0, keepdims=True)
    meta_ref[...] = jnp.concatenate(
        [tw, ew2, lo, hi, first_run, run_id, re_row, fetch_flag],
        axis=0).astype(jnp.int32)


def _route(x, w_router, interpret=False):
    s, d = x.shape
    e = w_router.shape[-1]
    return pl.pallas_call(
        _router_body,
        out_shape=[
            jax.ShapeDtypeStruct((s, e), jnp.float32),
            jax.ShapeDtypeStruct((s, 1), jnp.int32),
            jax.ShapeDtypeStruct((s, d), jnp.float32),
            jax.ShapeDtypeStruct((s // 128, 128), jnp.int32),
            jax.ShapeDtypeStruct((8, _MW), jnp.int32),
        ],
        interpret=interpret,
    )(x, w_router)


_NF = 1


def _ffn_body(meta_ref, xs_ref, wi_hbm, wo_hbm, y_ref,
              wib, wob, semi, semo):
    w = pl.program_id(0)
    run = meta_ref[5, w]
    par = lax.rem(run, 2)

    def _fetch_copies(expert, buf):
        d_h = wi_hbm.shape[1] // 2
        f_h = wo_hbm.shape[1] // 2
        return [
            pltpu.make_async_copy(wi_hbm.at[expert, pl.ds(0, d_h)],
                                  wib.at[buf, pl.ds(0, d_h)], semi.at[buf]),
            pltpu.make_async_copy(wi_hbm.at[expert, pl.ds(d_h, d_h)],
                                  wib.at[buf, pl.ds(d_h, d_h)], semi.at[buf]),
            pltpu.make_async_copy(wo_hbm.at[expert, pl.ds(0, f_h)],
                                  wob.at[buf, pl.ds(0, f_h)], semo.at[buf]),
            pltpu.make_async_copy(wo_hbm.at[expert, pl.ds(f_h, f_h)],
                                  wob.at[buf, pl.ds(f_h, f_h)], semo.at[buf]),
        ]

    @pl.when(w == 0)
    def _start():
        y_ref[...] = jnp.zeros_like(y_ref)
        cps = _fetch_copies(meta_ref[6, 0], 0)
        for cp in cps:
            cp.start()
        for cp in cps:
            cp.wait()

    @pl.when(meta_ref[7, w] == 1)
    def _prefetch():
        for cp in _fetch_copies(meta_ref[6, run + 1], 1 - par):
            cp.start()

    @pl.when((meta_ref[4, w] == 1) & (run > 0))
    def _wait():
        for cp in _fetch_copies(meta_ref[6, run], par):
            cp.wait()

    t = meta_ref[0, w]
    x = xs_ref[pl.ds(t * _T, _T), :]
    h = jnp.maximum(
        jnp.dot(x, wib[par], preferred_element_type=jnp.float32), 0.0)
    y = jnp.dot(h, wob[par], preferred_element_type=jnp.float32)
    rows = lax.broadcasted_iota(jnp.int32, (y.shape[0], 1), 0)
    mask = (rows >= meta_ref[2, w]) & (rows < meta_ref[3, w])
    y_ref[pl.ds(t * _T, _T), :] += jnp.where(mask, y, 0.0)


def _grouped_ffn(xs, wi, wo, meta, interpret=False):
    s, d = xs.shape
    e, _, f = wi.shape
    n_items = (s // _T) + e - 1
    grid_spec = pltpu.PrefetchScalarGridSpec(
        num_scalar_prefetch=1,
        grid=(n_items,),
        in_specs=[
            pl.BlockSpec((s, d), lambda w, meta: (0, 0)),
            pl.BlockSpec(memory_space=pl.ANY),
            pl.BlockSpec(memory_space=pl.ANY),
        ],
        out_specs=pl.BlockSpec((s, d), lambda w, meta: (0, 0)),
        scratch_shapes=[
            pltpu.VMEM((2, d, f), jnp.float32),
            pltpu.VMEM((2, f, d), jnp.float32),
            pltpu.SemaphoreType.DMA((2,)),
            pltpu.SemaphoreType.DMA((2,)),
        ],
    )
    return pl.pallas_call(
        _ffn_body,
        grid_spec=grid_spec,
        out_shape=jax.ShapeDtypeStruct((s, d), jnp.float32),
        compiler_params=pltpu.CompilerParams(
            dimension_semantics=("arbitrary",)),
        interpret=interpret,
    )(meta, xs, wi, wo)


def _sc_scatter_rows(src, pos):
    s, d = src.shape
    rpw = s // _NW
    mesh = plsc.VectorSubcoreMesh(core_axis_name="c", subcore_axis_name="s")

    @functools.partial(
        pl.kernel, mesh=mesh,
        out_type=jax.ShapeDtypeStruct((s, d), jnp.float32),
        scratch_types=[
            pltpu.VMEM((rpw,), jnp.int32),
            pltpu.VMEM((rpw, d), jnp.float32),
            pltpu.SemaphoreType.DMA,
        ])
    def k(src_hbm, idx_hbm, out_hbm, idx_v, rows_v, sem):
        wid = lax.axis_index("s") * 2 + lax.axis_index("c")
        base = wid * rpw
        pltpu.sync_copy(idx_hbm.at[pl.ds(base, rpw)], idx_v)
        pltpu.sync_copy(src_hbm.at[pl.ds(base, rpw)], rows_v)
        pltpu.async_copy(rows_v, out_hbm.at[idx_v], sem).wait()

    return k(src, pos)


def _sc_gather_rows(src, pos):
    s, d = src.shape
    rpw = s // _NW
    mesh = plsc.VectorSubcoreMesh(core_axis_name="c", subcore_axis_name="s")

    @functools.partial(
        pl.kernel, mesh=mesh,
        out_type=jax.ShapeDtypeStruct((s, d), jnp.float32),
        scratch_types=[
            pltpu.VMEM((rpw,), jnp.int32),
            pltpu.VMEM((rpw, d), jnp.float32),
            pltpu.SemaphoreType.DMA,
        ])
    def k(src_hbm, idx_hbm, out_hbm, idx_v, rows_v, sem):
        wid = lax.axis_index("s") * 2 + lax.axis_index("c")
        base = wid * rpw
        pltpu.sync_copy(idx_hbm.at[pl.ds(base, rpw)], idx_v)
        pltpu.async_copy(src_hbm.at[idx_v], rows_v, sem).wait()
        pltpu.sync_copy(rows_v, out_hbm.at[pl.ds(base, rpw)])

    return k(src, pos)


def kernel(hidden_states, W_router, wi, wo):
    b, s, d = hidden_states.shape
    e = W_router.shape[-1]
    x = hidden_states.reshape(b * s, d)

    logits, idx2, x_scaled, pos2, meta = _route(x, W_router)
    pos = pos2.reshape(b * s)

    xs = _sc_scatter_rows(x_scaled, pos)
    ys = _grouped_ffn(xs, wi, wo, meta)
    out = _sc_gather_rows(ys, pos)

    return (out.reshape(b, s, d),
            logits.reshape(b, s, e),
            idx2.reshape(b, s))
